# baseline (speedup 1.0000x reference)
import functools
import math

import jax
import jax.numpy as jnp
from jax import lax
from jax.experimental import pallas as pl
from jax.experimental.pallas import tpu as pltpu
from jax.experimental.pallas import tpu_sc as plsc

D_EMBED = 64
SCALE = math.sqrt(D_EMBED)
L = 16
HC = 256
NBUF = 2


def _build_sc_kernel(num_rows_x, num_cols_x):
    try:
        info = plsc.get_sparse_core_info()
        nc, ns = info.num_cores, info.num_subcores
    except Exception:
        nc, ns = 2, 16
    nw = nc * ns
    rows_w = num_rows_x // nw
    assert rows_w * nw == num_rows_x and rows_w & (rows_w - 1) == 0
    assert rows_w % HC == 0
    nh = rows_w // HC
    per_w = rows_w * num_cols_x
    nchunk = per_w // HC
    assert nchunk % NBUF == 0 and nchunk >= 2 * NBUF
    rbits = rows_w.bit_length() - 1

    mesh = plsc.VectorSubcoreMesh(core_axis_name="c", subcore_axis_name="s")

    @functools.partial(
        pl.kernel,
        mesh=mesh,
        compiler_params=pltpu.CompilerParams(
            use_tc_tiling_on_sc=False, needs_layout_passes=False),
        out_type=jax.ShapeDtypeStruct(
            (num_cols_x, D_EMBED, num_rows_x), jnp.float32),
        scratch_types=(
            [pltpu.VMEM((rows_w, num_cols_x), jnp.int32),
             pltpu.VMEM((per_w,), jnp.int32)]
            + [pltpu.VMEM((HC, D_EMBED), jnp.float32) for _ in range(NBUF)]
            + [pltpu.VMEM((D_EMBED, HC), jnp.float32) for _ in range(NBUF)]
            + [pltpu.SemaphoreType.DMA for _ in range(2 * NBUF)]
        ),
    )
    def emb(x_hbm, table_hbm, out_hbm, xstage, xidx, *bufs_and_sems):
        gbuf = bufs_and_sems[0:NBUF]
        sbuf = bufs_and_sems[NBUF:2 * NBUF]
        gsem = bufs_and_sems[2 * NBUF:3 * NBUF]
        ssem = bufs_and_sems[3 * NBUF:4 * NBUF]

        wid = lax.axis_index("s") * nc + lax.axis_index("c")
        i0 = wid * rows_w

        pltpu.sync_copy(x_hbm.at[pl.ds(i0, rows_w)], xstage)

        lane_iota = jax.lax.iota(jnp.int32, L)

        @plsc.parallel_loop(0, per_w // L, unroll=4)
        def _(t):
            q = lane_iota + t * L
            j = jax.lax.shift_right_logical(q, rbits)
            r = jax.lax.bitwise_and(q, rows_w - 1)
            xidx[pl.ds(t * L, L)] = plsc.load_gather(xstage, [r, j])

        def start_gather(c, b):
            pltpu.async_copy(
                table_hbm.at[xidx.at[pl.ds(c * HC, HC)]], gbuf[b], gsem[b])

        def wait_gather(b):
            pltpu.make_async_copy(
                table_hbm.at[xidx.at[pl.ds(0, HC)]], gbuf[b], gsem[b]).wait()

        def start_scatter(c, b):
            j, h = c // nh, c % nh
            pltpu.async_copy(
                sbuf[b], out_hbm.at[j, :, pl.ds(i0 + h * HC, HC)], ssem[b])

        def wait_scatter(b):
            pltpu.make_async_copy(
                sbuf[b], out_hbm.at[0, :, pl.ds(i0, HC)], ssem[b]).wait()

        def transpose_scale(b):
            gb, sb = gbuf[b], sbuf[b]

            @plsc.parallel_loop(0, D_EMBED, unroll=2)
            def _(d):
                dv = jnp.zeros((L,), jnp.int32) + d
                for kk in range(HC // L):
                    k = lane_iota + kk * L
                    v = plsc.load_gather(gb, [k, dv])
                    sb[d, pl.ds(kk * L, L)] = v * SCALE

        for b in range(NBUF):
            start_gather(b, b)

        for b in range(NBUF):
            wait_gather(b)
            transpose_scale(b)
            start_scatter(b, b)
            start_gather(b + NBUF, b)

        @pl.loop(NBUF, nchunk - NBUF, step=NBUF)
        def _(g):
            for b in range(NBUF):
                c = g + b
                wait_gather(b)
                wait_scatter(b)
                transpose_scale(b)
                start_scatter(c, b)
                start_gather(c + NBUF, b)

        for b in range(NBUF):
            c = nchunk - NBUF + b
            wait_gather(b)
            wait_scatter(b)
            transpose_scale(b)
            start_scatter(c, b)

        for b in range(NBUF):
            wait_scatter(b)

    return emb


def kernel(x, table):
    rows, cols = x.shape
    emb = _build_sc_kernel(rows, cols)
    out_t = emb(x, table)
    return out_t.transpose(2, 0, 1)
